# baseline (speedup 1.0000x reference)
import functools

import jax
import jax.numpy as jnp
from jax import lax
from jax.experimental import pallas as pl
from jax.experimental.pallas import tpu as pltpu
from jax.experimental.pallas import tpu_sc as plsc

_N_ITERS = 5
_ROW_BLOCK = 1024
_NUM_CORES = 2
_NUM_SUBCORES = 16
_LANES = 16


def _dense_kernel(x_ref, out_ref, *, C: int):
    R = x_ref.shape[0]
    x = x_ref[...]
    col = jax.lax.broadcasted_iota(jnp.int32, (R, x.shape[1]), 1)
    valid = col < C
    x = jnp.where(valid, x, 0.0)
    neg_big = jnp.float32(-1e30)
    mx = jnp.max(jnp.where(valid, x, neg_big), axis=1, keepdims=True)
    Xs = jnp.where(valid, (x - mx) * 0.5, neg_big)

    tau = jnp.full((R, 1), -1.0, dtype=jnp.float32)
    for _ in range(_N_ITERS):
        r = jnp.maximum(Xs - tau, 0.0)
        s1 = jnp.sum(r, axis=1, keepdims=True)
        s2 = jnp.sum(r * r, axis=1, keepdims=True)
        k = jnp.sum((Xs > tau).astype(jnp.float32), axis=1, keepdims=True)
        ss = s2 - s1 * s1 / k
        delta = jnp.maximum((1.0 - ss) / k, 0.0)
        tau = tau + s1 / k - jnp.sqrt(delta)

    r = jnp.maximum(Xs - tau, 0.0)
    row_loss = ((1.0 - jnp.sum(r * r * r, axis=1)) / 0.75
                + jnp.sum(r * r * x, axis=1))
    block_sum = jnp.sum(row_loss).reshape(1, 1)

    @pl.when(pl.program_id(0) == 0)
    def _():
        out_ref[...] = jnp.zeros((1, 1), jnp.float32)

    out_ref[...] += block_sum


_GRAN = 128


def _sc_gather_granules(x128, tgt, n: int, C: int):
    NW = _NUM_CORES * _NUM_SUBCORES
    chunk = n // NW
    nb = chunk // _LANES

    mesh = plsc.VectorSubcoreMesh(core_axis_name="c", subcore_axis_name="s")

    @functools.partial(
        pl.kernel, mesh=mesh,
        out_type=jax.ShapeDtypeStruct((n, _GRAN), jnp.float32),
        scratch_types=[
            pltpu.VMEM((chunk,), jnp.int32),
            pltpu.VMEM((chunk,), jnp.int32),
            pltpu.VMEM((chunk, _GRAN), jnp.float32),
            pltpu.SemaphoreType.DMA,
        ],
    )
    def sc_gather(x_hbm, tgt_hbm, out_hbm, tgt_v, idx_v, rows_v, sem):
        wid = lax.axis_index("s") * _NUM_CORES + lax.axis_index("c")
        base = wid * chunk
        pltpu.sync_copy(tgt_hbm.at[pl.ds(base, chunk)], tgt_v)
        iota = lax.iota(jnp.int32, _LANES)
        for b in range(nb):
            t16 = tgt_v[pl.ds(b * _LANES, _LANES)]
            p16 = (base + b * _LANES) * C + iota * C + t16
            idx_v[pl.ds(b * _LANES, _LANES)] = p16 >> 7
        pltpu.async_copy(x_hbm.at[idx_v], rows_v, sem).wait()
        pltpu.sync_copy(rows_v, out_hbm.at[pl.ds(base, chunk)])

    return sc_gather(x128, tgt)


def _select_kernel(g_ref, tgt_ref, out_ref, *, C: int, R2: int):
    blk = pl.program_id(0)
    tgt = tgt_ref[0, 0, :].reshape(R2, 1)
    row = (jax.lax.broadcasted_iota(jnp.int32, (R2, 1), 0) + blk * R2)
    lane_want = jax.lax.rem(row * C + tgt, jnp.int32(_GRAN))
    lane = jax.lax.broadcasted_iota(jnp.int32, (R2, _GRAN), 1)
    vals = jnp.where(lane == lane_want, g_ref[...], 0.0)
    block_sum = jnp.sum(vals).reshape(1, 1)

    @pl.when(blk == 0)
    def _():
        out_ref[...] = jnp.zeros((1, 1), jnp.float32)

    out_ref[...] += block_sum


@jax.jit
def kernel(input, target):
    n, C = input.shape
    R = _ROW_BLOCK
    nb = n // R
    tgt = target.astype(jnp.int32)
    x128 = input.reshape(n * C // _GRAN, _GRAN)
    granules = _sc_gather_granules(x128, tgt, n, C)
    total = pl.pallas_call(
        functools.partial(_dense_kernel, C=C),
        grid=(nb,),
        in_specs=[pl.BlockSpec((R, C), lambda i: (i, 0))],
        out_specs=pl.BlockSpec((1, 1), lambda i: (0, 0)),
        out_shape=jax.ShapeDtypeStruct((1, 1), jnp.float32),
    )(input)
    R2 = 2048
    nb2 = n // R2
    tgt3 = tgt.reshape(nb2, 1, R2)
    tgt_sum = pl.pallas_call(
        functools.partial(_select_kernel, C=C, R2=R2),
        grid=(nb2,),
        in_specs=[
            pl.BlockSpec((R2, _GRAN), lambda i: (i, 0)),
            pl.BlockSpec((1, 1, R2), lambda i: (i, 0, 0)),
        ],
        out_specs=pl.BlockSpec((1, 1), lambda i: (0, 0)),
        out_shape=jax.ShapeDtypeStruct((1, 1), jnp.float32),
    )(granules, tgt3)
    return (total[0, 0] - tgt_sum[0, 0]) / jnp.float32(n)


# speedup vs baseline: 1.2682x; 1.2682x over previous
import functools

import jax
import jax.numpy as jnp
from jax.experimental import pallas as pl

_N_ITERS = 5
_ROW_BLOCK = 1024


def _loss_kernel(x_ref, tgt_ref, out_ref, *, C: int):
    R = x_ref.shape[0]
    x = x_ref[...]
    col = jax.lax.broadcasted_iota(jnp.int32, (R, x.shape[1]), 1)
    valid = col < C
    neg_big = jnp.float32(-1e30)
    mx = jnp.max(jnp.where(valid, x, neg_big), axis=1, keepdims=True)
    Xs = jnp.where(valid, (x - mx) * 0.5, neg_big)

    tau = jnp.full((R, 1), -1.0, dtype=jnp.float32)
    for _ in range(_N_ITERS):
        r = jnp.maximum(Xs - tau, 0.0)
        s1 = jnp.sum(r, axis=1, keepdims=True)
        s2 = jnp.sum(r * r, axis=1, keepdims=True)
        k = jnp.sum(jnp.sign(r), axis=1, keepdims=True)
        ss = s2 - s1 * s1 / k
        delta = jnp.maximum((1.0 - ss) / k, 0.0)
        tau = tau + s1 / k - jnp.sqrt(delta)

    r = jnp.maximum(Xs - tau, 0.0)
    r2 = r * r
    s2 = jnp.sum(r2, axis=1, keepdims=True)
    s3 = jnp.sum(r2 * r, axis=1, keepdims=True)
    tgt = tgt_ref[0, 0, :].reshape(R, 1)
    xs_t = jnp.sum(jnp.where(col == tgt, Xs, 0.0), axis=1, keepdims=True)
    row_loss = ((1.0 - s3) / 0.75 + 2.0 * s3 + (2.0 * tau + mx) * s2
                - (2.0 * xs_t + mx))
    block_sum = jnp.sum(row_loss).reshape(1, 1)

    @pl.when(pl.program_id(0) == 0)
    def _():
        out_ref[...] = jnp.zeros((1, 1), jnp.float32)

    out_ref[...] += block_sum


@jax.jit
def kernel(input, target):
    n, C = input.shape
    R = _ROW_BLOCK
    nb = n // R
    tgt3 = target.astype(jnp.int32).reshape(nb, 1, R)
    total = pl.pallas_call(
        functools.partial(_loss_kernel, C=C),
        grid=(nb,),
        in_specs=[
            pl.BlockSpec((R, C), lambda i: (i, 0)),
            pl.BlockSpec((1, 1, R), lambda i: (i, 0, 0)),
        ],
        out_specs=pl.BlockSpec((1, 1), lambda i: (0, 0)),
        out_shape=jax.ShapeDtypeStruct((1, 1), jnp.float32),
    )(input, tgt3)
    return total[0, 0] / jnp.float32(n)


# speedup vs baseline: 1.6467x; 1.2984x over previous
import functools

import jax
import jax.numpy as jnp
from jax.experimental import pallas as pl

_N_ITERS = 4
_ROW_BLOCK = 1024


def _loss_kernel(x_ref, tgt_ref, out_ref, *, C: int):
    R = x_ref.shape[0]
    x = x_ref[...]
    col = jax.lax.broadcasted_iota(jnp.int32, (R, x.shape[1]), 1)
    valid = col < C
    neg_big = jnp.float32(-1e30)
    mx = jnp.max(jnp.where(valid, x, neg_big), axis=1, keepdims=True)
    Xs = jnp.where(valid, (x - mx) * 0.5, neg_big)

    tau = jnp.full((R, 1), -1.0, dtype=jnp.float32)
    for _ in range(_N_ITERS):
        r = jnp.maximum(Xs - tau, 0.0)
        s1 = jnp.sum(r, axis=1, keepdims=True)
        s2 = jnp.sum(r * r, axis=1, keepdims=True)
        k = jnp.sum((Xs > tau).astype(jnp.float32), axis=1, keepdims=True)
        ss = s2 - s1 * s1 / k
        delta = jnp.maximum((1.0 - ss) / k, 0.0)
        tau = tau + s1 / k - jnp.sqrt(delta)

    r = jnp.maximum(Xs - tau, 0.0)
    r2 = r * r
    s2 = jnp.sum(r2, axis=1, keepdims=True)
    s3 = jnp.sum(r2 * r, axis=1, keepdims=True)
    tgt = tgt_ref[0, 0, :].reshape(R, 1)
    xs_t = jnp.sum(jnp.where(col == tgt, Xs, 0.0), axis=1, keepdims=True)
    row_loss = ((1.0 - s3) / 0.75 + 2.0 * s3 + (2.0 * tau + mx) * s2
                - (2.0 * xs_t + mx))
    block_sum = jnp.sum(row_loss).reshape(1, 1)

    @pl.when(pl.program_id(0) == 0)
    def _():
        out_ref[...] = jnp.zeros((1, 1), jnp.float32)

    out_ref[...] += block_sum


@jax.jit
def kernel(input, target):
    n, C = input.shape
    R = _ROW_BLOCK
    nb = n // R
    tgt3 = target.astype(jnp.int32).reshape(nb, 1, R)
    total = pl.pallas_call(
        functools.partial(_loss_kernel, C=C),
        grid=(nb,),
        in_specs=[
            pl.BlockSpec((R, C), lambda i: (i, 0)),
            pl.BlockSpec((1, 1, R), lambda i: (i, 0, 0)),
        ],
        out_specs=pl.BlockSpec((1, 1), lambda i: (0, 0)),
        out_shape=jax.ShapeDtypeStruct((1, 1), jnp.float32),
    )(input, tgt3)
    return total[0, 0] / jnp.float32(n)
